# baseline (speedup 1.0000x reference)
import functools

import jax
import jax.numpy as jnp
from jax import lax
from jax.experimental import pallas as pl
from jax.experimental.pallas import tpu as pltpu
from jax.experimental.pallas import tpu_sc as plsc

D_GLYPH = 512
D = 1024
BATCH = 4
SEQ = 2048
TOK = BATCH * SEQ
NC = 2
NS = 16
NW = NC * NS
TPW = TOK // NW
CH = 16
NCH = TPW // CH
LN_EPS = 1e-12
VPG = D_GLYPH // 16
RSQRT_MAGIC = 0x5F3759DF


def _sc_fused(ids, glyph, graph, stroke, pos, gamma, beta):
  mesh = plsc.VectorSubcoreMesh(core_axis_name="c", subcore_axis_name="s")

  @functools.partial(
      pl.kernel,
      mesh=mesh,
      compiler_params=pltpu.CompilerParams(needs_layout_passes=False),
      out_type=[
          jax.ShapeDtypeStruct((TOK, D), jnp.float32),
          jax.ShapeDtypeStruct((NW, 16), jnp.float32),
      ],
      scratch_types=[
          pltpu.VMEM((TPW,), jnp.int32),
          pltpu.VMEM((D,), jnp.float32),
          pltpu.VMEM((D,), jnp.float32),
          pltpu.VMEM((CH, D_GLYPH), jnp.float32),
          pltpu.VMEM((CH, D_GLYPH), jnp.float32),
          pltpu.VMEM((CH, D), jnp.float32),
          pltpu.VMEM((CH, D), jnp.float32),
          pltpu.VMEM((CH, D_GLYPH), jnp.float32),
          pltpu.VMEM((CH, D_GLYPH), jnp.float32),
          pltpu.VMEM((CH, D), jnp.float32),
          pltpu.VMEM((CH, D), jnp.float32),
          pltpu.VMEM((16,), jnp.float32),
          pltpu.VMEM((8, 32), jnp.float32),
          pltpu.SemaphoreType.DMA,
          pltpu.SemaphoreType.DMA,
          pltpu.SemaphoreType.DMA,
          pltpu.SemaphoreType.DMA,
      ],
  )
  def body(ids_hbm, glyph_hbm, graph_hbm, stroke_hbm, pos_hbm, gam_hbm,
           bet_hbm, out_hbm, sq_hbm, idx_all, gam_v, bet_v,
           gly0, gra0, str0, pv0, gly1, gra1, str1, pv1,
           sq_v, red_v, semg0, semg1, semo0, semo1):
    wid = lax.axis_index("s") * NC + lax.axis_index("c")
    base = wid * TPW
    pltpu.sync_copy(ids_hbm.at[pl.ds(base, TPW)], idx_all)
    pltpu.sync_copy(gam_hbm, gam_v)
    pltpu.sync_copy(bet_hbm, bet_v)
    bufs = ((gly0, gra0, str0, pv0, semg0, semo0),
            (gly1, gra1, str1, pv1, semg1, semo1))

    def gather_parts(k, b):
      gly, gra, stv, pv, semg, _ = bufs[b]
      idx = idx_all.at[pl.ds(k * CH, CH)]
      s0 = lax.rem(base + k * CH, SEQ)
      return ((glyph_hbm.at[idx], gly, semg),
              (graph_hbm.at[idx], gra, semg),
              (stroke_hbm.at[idx], stv, semg),
              (pos_hbm.at[pl.ds(s0, CH)], pv, semg))

    def fire_tables(k, b):
      for src, dst, sem in gather_parts(k, b)[:3]:
        pltpu.async_copy(src, dst, sem)

    def fire_pos(k, b):
      src, dst, sem = gather_parts(k, b)[3]
      pltpu.async_copy(src, dst, sem)

    def wait_gathers(k, b):
      for src, dst, sem in gather_parts(k, b):
        pltpu.make_async_copy(src, dst, sem).wait()

    def wait_out(k, b):
      _, _, _, pv, _, semo = bufs[b]
      pltpu.make_async_copy(
          pv, out_hbm.at[pl.ds(base + k * CH, CH)], semo).wait()

    def rsqrt16(x16):
      bits = plsc.bitcast(x16, jnp.int32)
      magic = jnp.full((16,), RSQRT_MAGIC, dtype=jnp.int32)
      y = plsc.bitcast(magic - lax.shift_right_logical(bits, 1),
                       jnp.float32)
      for _ in range(3):
        y = y * (1.5 - 0.5 * x16 * y * y)
      return y

    def compute(k, b, accs):
      gly, gra, stv, pv, _, semo = bufs[b]

      def tgroup(tg, accs):
        t0 = tg * 4
        zz = jnp.zeros((16,), jnp.float32)

        def pass1(jj, carry):
          s4 = list(carry[0:4])
          q4 = list(carry[4:8])
          ga = list(carry[8:12])
          for u in range(2):
            o = (jj * 2 + u) * 16
            gs = [gly[t0 + dt, pl.ds(o, 16)] for dt in range(4)]
            sl = [stv[t0 + dt, pl.ds(o, 16)] for dt in range(4)]
            plv = [pv[t0 + dt, pl.ds(o, 16)] for dt in range(4)]
            rs = [gra[t0 + dt, pl.ds(o, 16)] for dt in range(4)]
            sr = [stv[t0 + dt, pl.ds(D_GLYPH + o, 16)] for dt in range(4)]
            prv = [pv[t0 + dt, pl.ds(D_GLYPH + o, 16)] for dt in range(4)]
            for dt in range(4):
              el = sl[dt] + gs[dt] + plv[dt]
              er = sr[dt] + rs[dt] + prv[dt]
              stv[t0 + dt, pl.ds(o, 16)] = el
              stv[t0 + dt, pl.ds(D_GLYPH + o, 16)] = er
              s4[dt] = s4[dt] + (el + er)
              q4[dt] = q4[dt] + el * el
              q4[dt] = q4[dt] + er * er
              ga[dt] = ga[dt] + gs[dt] * gs[dt]
          return tuple(s4) + tuple(q4) + tuple(ga)

        carry = (zz,) * 8 + tuple(accs)
        carry = lax.fori_loop(0, VPG // 2, pass1, carry)
        accs = carry[8:12]

        vs = list(carry[0:8])
        for i in range(8):
          red_v[i, pl.ds(0, 16)] = vs[i]
        for sh in (8, 4, 2, 1):
          vs = [vs[i] + red_v[i, pl.ds(sh, 16)] for i in range(8)]
          for i in range(8):
            red_v[i, pl.ds(0, 16)] = vs[i]
        ms = []
        ws = []
        for dt in range(4):
          m16 = jnp.full((16,), red_v[dt, pl.ds(0, 16)][0],
                         jnp.float32) * (1.0 / D)
          q16 = jnp.full((16,), red_v[4 + dt, pl.ds(0, 16)][0],
                         jnp.float32) * (1.0 / D)
          var = q16 - m16 * m16
          ms.append(m16)
          ws.append(rsqrt16(var + LN_EPS))

        def pass2(jj, carry2):
          for u in range(2):
            o = (jj * 2 + u) * 16
            gl = gam_v[pl.ds(o, 16)]
            bl = bet_v[pl.ds(o, 16)]
            gr = gam_v[pl.ds(D_GLYPH + o, 16)]
            br = bet_v[pl.ds(D_GLYPH + o, 16)]
            el = [stv[t0 + dt, pl.ds(o, 16)] for dt in range(4)]
            er = [stv[t0 + dt, pl.ds(D_GLYPH + o, 16)] for dt in range(4)]
            for dt in range(4):
              pv[t0 + dt, pl.ds(o, 16)] = (
                  (el[dt] - ms[dt]) * ws[dt] * gl + bl)
              pv[t0 + dt, pl.ds(D_GLYPH + o, 16)] = (
                  (er[dt] - ms[dt]) * ws[dt] * gr + br)
          return carry2

        lax.fori_loop(0, VPG // 2, pass2, jnp.int32(0))
        return accs

      accs = lax.fori_loop(0, CH // 4, tgroup, accs)
      pltpu.async_copy(pv, out_hbm.at[pl.ds(base + k * CH, CH)], semo)
      return accs

    fire_tables(0, 0)
    fire_pos(0, 0)

    def pair(g, accs):
      k0 = 2 * g
      fire_tables(k0 + 1, 1)

      @pl.when(g > 0)
      def _():
        wait_out(k0 - 1, 1)

      fire_pos(k0 + 1, 1)
      wait_gathers(k0, 0)
      accs = compute(k0, 0, accs)

      @pl.when(g < NCH // 2 - 1)
      def _():
        fire_tables(k0 + 2, 0)
        wait_out(k0, 0)
        fire_pos(k0 + 2, 0)

      wait_gathers(k0 + 1, 1)
      accs = compute(k0 + 1, 1, accs)
      return accs

    accs = lax.fori_loop(0, NCH // 2, pair,
                         (jnp.zeros((16,), jnp.float32),) * 4)
    wait_out(NCH - 2, 0)
    wait_out(NCH - 1, 1)
    sq_v[...] = accs[0] + accs[1] + accs[2] + accs[3]
    pltpu.sync_copy(sq_v, sq_hbm.at[wid])

  return body(ids, glyph, graph, stroke, pos, gamma, beta)


def _loss_body(sq_ref, loss_ref):
  loss_ref[...] = (jnp.sum(sq_ref[...]) / float(TOK * D_GLYPH)).reshape(1, 1)


def kernel(input_ids, pos_table, glyph_table, graph_table, stroke_table,
           gamma, beta):
  ids = input_ids.astype(jnp.int32).reshape(TOK)
  emb, partials = _sc_fused(
      ids, glyph_table, graph_table, stroke_table, pos_table, gamma, beta)

  loss = pl.pallas_call(
      _loss_body,
      out_shape=jax.ShapeDtypeStruct((1, 1), jnp.float32),
  )(partials)

  return emb.reshape(BATCH, SEQ, D), loss[0, 0]
